# baseline (speedup 1.0000x reference)
import jax
import jax.numpy as jnp
from jax import lax
from jax.experimental import pallas as pl

_EMB = 128
_BB = 8


def _body(rand_ref, caps_ref, w_ref, b_ref, row_ref, col_ref, caps_out_ref):
    bb, c = rand_ref.shape
    r = rand_ref[...]
    rj = r[:, :, None]
    rk = r[:, None, :]
    j_ids = lax.broadcasted_iota(jnp.int32, (bb, c, c), 1)
    k_ids = lax.broadcasted_iota(jnp.int32, (bb, c, c), 2)
    before = (rk < rj) | ((rk == rj) & (k_ids < j_ids))
    rank = jnp.sum(before.astype(jnp.int32), axis=2)
    n_ids = lax.broadcasted_iota(jnp.int32, (bb, c, c), 1)
    col_ref[...] = (rank[:, None, :] == n_ids).astype(jnp.float32)
    row_ref[...] = jnp.zeros(row_ref.shape, row_ref.dtype)
    acc = lax.dot_general(
        caps_ref[...], w_ref[...], (((1,), (1,)), ((), ())),
        preferred_element_type=jnp.float32,
        precision=lax.Precision.HIGHEST,
    )
    caps_out_ref[...] = acc + b_ref[...]


def kernel(cost_matrix, node_capacities, W, b):
    bsz, r, c = cost_matrix.shape
    m = node_capacities.shape[1]
    rand = jax.random.uniform(jax.random.key(42), (bsz, c))
    b2 = b.reshape(1, r)
    grid = bsz // _BB
    row_emb, col_emb, caps_out = pl.pallas_call(
        _body,
        grid=(grid,),
        in_specs=[
            pl.BlockSpec((_BB, c), lambda i: (i, 0)),
            pl.BlockSpec((_BB, m), lambda i: (i, 0)),
            pl.BlockSpec((r, m), lambda i: (0, 0)),
            pl.BlockSpec((1, r), lambda i: (0, 0)),
        ],
        out_specs=[
            pl.BlockSpec((_BB, r, _EMB), lambda i: (i, 0, 0)),
            pl.BlockSpec((_BB, c, _EMB), lambda i: (i, 0, 0)),
            pl.BlockSpec((_BB, r), lambda i: (i, 0)),
        ],
        out_shape=[
            jax.ShapeDtypeStruct((bsz, r, _EMB), cost_matrix.dtype),
            jax.ShapeDtypeStruct((bsz, c, _EMB), cost_matrix.dtype),
            jax.ShapeDtypeStruct((bsz, r), jnp.float32),
        ],
    )(rand, node_capacities, W, b2)
    return (row_emb, col_emb, cost_matrix, caps_out)


# speedup vs baseline: 8.7187x; 8.7187x over previous
import jax
import jax.numpy as jnp
from jax import lax
from jax.experimental import pallas as pl

_EMB = 128
_BB = 8


def _body(rand_ref, caps_ref, w_ref, b_ref, row_ref, col_ref, caps_out_ref):
    bb, c = rand_ref.shape
    k_sub = lax.broadcasted_iota(jnp.int32, (c, c), 0)
    j_lane = lax.broadcasted_iota(jnp.int32, (c, c), 1)
    tri = k_sub < j_lane
    n_sub = k_sub
    r_all = rand_ref[...]
    rt_all = jnp.transpose(r_all)
    for i in range(bb):
        rj = r_all[i:i + 1, :]
        rk = rt_all[:, i:i + 1]
        before = (rk < rj) | ((rk == rj) & tri)
        rank = jnp.sum(before.astype(jnp.int32), axis=0, keepdims=True)
        col_ref[i] = (n_sub == rank).astype(jnp.float32)
    row_ref[...] = jnp.zeros(row_ref.shape, row_ref.dtype)
    acc = lax.dot_general(
        caps_ref[...], w_ref[...], (((1,), (1,)), ((), ())),
        preferred_element_type=jnp.float32,
        precision=lax.Precision.HIGHEST,
    )
    caps_out_ref[...] = acc + b_ref[...]


def kernel(cost_matrix, node_capacities, W, b):
    bsz, r, c = cost_matrix.shape
    m = node_capacities.shape[1]
    rand = jax.random.uniform(jax.random.key(42), (bsz, c))
    b2 = b.reshape(1, r)
    grid = bsz // _BB
    row_emb, col_emb, caps_out = pl.pallas_call(
        _body,
        grid=(grid,),
        in_specs=[
            pl.BlockSpec((_BB, c), lambda i: (i, 0)),
            pl.BlockSpec((_BB, m), lambda i: (i, 0)),
            pl.BlockSpec((r, m), lambda i: (0, 0)),
            pl.BlockSpec((1, r), lambda i: (0, 0)),
        ],
        out_specs=[
            pl.BlockSpec((_BB, r, _EMB), lambda i: (i, 0, 0)),
            pl.BlockSpec((_BB, c, _EMB), lambda i: (i, 0, 0)),
            pl.BlockSpec((_BB, r), lambda i: (i, 0)),
        ],
        out_shape=[
            jax.ShapeDtypeStruct((bsz, r, _EMB), cost_matrix.dtype),
            jax.ShapeDtypeStruct((bsz, c, _EMB), cost_matrix.dtype),
            jax.ShapeDtypeStruct((bsz, r), jnp.float32),
        ],
    )(rand, node_capacities, W, b2)
    return (row_emb, col_emb, cost_matrix, caps_out)


# speedup vs baseline: 11.4775x; 1.3164x over previous
import jax
import jax.numpy as jnp
from jax import lax
from jax.experimental import pallas as pl

_EMB = 128
_BB = 32


def _body(rand_ref, caps_ref, w_ref, b_ref, row_ref, col_ref, caps_out_ref):
    bb, c = rand_ref.shape
    k_sub = lax.broadcasted_iota(jnp.int32, (c, c), 0)
    j_lane = lax.broadcasted_iota(jnp.int32, (c, c), 1)
    tri = k_sub < j_lane
    n_sub = k_sub
    r_all = rand_ref[...]
    rt_all = jnp.transpose(r_all)
    for i in range(bb):
        rj = r_all[i:i + 1, :]
        rk = rt_all[:, i:i + 1]
        before = (rk < rj) | ((rk == rj) & tri)
        rank = jnp.sum(before.astype(jnp.int32), axis=0, keepdims=True)
        col_ref[i] = (n_sub == rank).astype(jnp.float32)
    row_ref[...] = jnp.zeros(row_ref.shape, row_ref.dtype)
    acc = lax.dot_general(
        caps_ref[...], w_ref[...], (((1,), (1,)), ((), ())),
        preferred_element_type=jnp.float32,
        precision=lax.Precision.HIGHEST,
    )
    caps_out_ref[...] = acc + b_ref[...]


def kernel(cost_matrix, node_capacities, W, b):
    bsz, r, c = cost_matrix.shape
    m = node_capacities.shape[1]
    rand = jax.random.uniform(jax.random.key(42), (bsz, c))
    b2 = b.reshape(1, r)
    grid = bsz // _BB
    row_emb, col_emb, caps_out = pl.pallas_call(
        _body,
        grid=(grid,),
        in_specs=[
            pl.BlockSpec((_BB, c), lambda i: (i, 0)),
            pl.BlockSpec((_BB, m), lambda i: (i, 0)),
            pl.BlockSpec((r, m), lambda i: (0, 0)),
            pl.BlockSpec((1, r), lambda i: (0, 0)),
        ],
        out_specs=[
            pl.BlockSpec((_BB, r, _EMB), lambda i: (i, 0, 0)),
            pl.BlockSpec((_BB, c, _EMB), lambda i: (i, 0, 0)),
            pl.BlockSpec((_BB, r), lambda i: (i, 0)),
        ],
        out_shape=[
            jax.ShapeDtypeStruct((bsz, r, _EMB), cost_matrix.dtype),
            jax.ShapeDtypeStruct((bsz, c, _EMB), cost_matrix.dtype),
            jax.ShapeDtypeStruct((bsz, r), jnp.float32),
        ],
    )(rand, node_capacities, W, b2)
    return (row_emb, col_emb, cost_matrix, caps_out)
